# baseline (device time: 39693 ns/iter reference)
import jax
import jax.numpy as jnp
from jax import lax
from jax.experimental import pallas as pl
from jax.experimental.pallas import tpu as pltpu

N_DEV = 16
B, SQ, D, HLOC, DH, SKV = 2, 128, 512, 8, 64, 128
ROWS = B * SQ
CHUNK = ROWS // N_DEV


def kernel(x, Wq, Wo, K_ext, V_ext):
    my = lax.axis_index("i")
    K_loc = lax.dynamic_slice(K_ext, (0, 0, my * HLOC, 0), (B, SKV, HLOC, DH))
    V_loc = lax.dynamic_slice(V_ext, (0, 0, my * HLOC, 0), (B, SKV, HLOC, DH))
    K_loc = jnp.transpose(K_loc, (0, 2, 1, 3)).reshape(B * HLOC, SKV, DH)
    V_loc = jnp.transpose(V_loc, (0, 2, 1, 3)).reshape(B * HLOC, SKV, DH)
    x2 = x.reshape(ROWS, D)

    def body(x_ref, wq_ref, wo_ref, k_ref, v_ref, out_ref,
             attn_ref, partial_ref, red_ref, comm_ref,
             send1, recv1, send2, recv2):
        my_pos = lax.axis_index("i")

        bar = pltpu.get_barrier_semaphore()
        for d in range(1, N_DEV):
            pl.semaphore_signal(
                bar, inc=1,
                device_id=((my_pos + d) % N_DEV,),
                device_id_type=pl.DeviceIdType.MESH,
            )
        pl.semaphore_wait(bar, N_DEV - 1)

        q = jnp.dot(x_ref[...], wq_ref[...], preferred_element_type=jnp.float32)

        for b in range(B):
            for h in range(HLOC):
                bh = b * HLOC + h
                qbh = q[b * SQ:(b + 1) * SQ, h * DH:(h + 1) * DH]
                kbh = k_ref[bh]
                vbh = v_ref[bh]
                s = lax.dot_general(
                    qbh, kbh, (((1,), (1,)), ((), ())),
                    preferred_element_type=jnp.float32,
                ) * 0.125
                m = jnp.max(s, axis=1, keepdims=True)
                p = jnp.exp(s - m)
                l = jnp.sum(p, axis=1, keepdims=True)
                o = lax.dot_general(
                    p, vbh, (((1,), (0,)), ((), ())),
                    preferred_element_type=jnp.float32,
                ) / l
                attn_ref[b * SQ:(b + 1) * SQ, h * DH:(h + 1) * DH] = o

        partial_ref[...] = jnp.dot(
            attn_ref[...], wo_ref[...], preferred_element_type=jnp.float32
        )

        sends1 = []
        for d in range(1, N_DEV):
            tgt = (my_pos + d) % N_DEV
            rd = pltpu.make_async_remote_copy(
                src_ref=partial_ref.at[pl.ds(tgt * CHUNK, CHUNK), :],
                dst_ref=comm_ref.at[d],
                send_sem=send1.at[d],
                recv_sem=recv1.at[d],
                device_id=(tgt,),
                device_id_type=pl.DeviceIdType.MESH,
            )
            rd.start()
            sends1.append(rd)

        acc = partial_ref[pl.ds(my_pos * CHUNK, CHUNK), :]
        for d in range(1, N_DEV):
            sends1[d - 1].wait_recv()
            acc = acc + comm_ref[d]
        red_ref[...] = acc

        sends2 = []
        for d in range(1, N_DEV):
            tgt = (my_pos + d) % N_DEV
            rd = pltpu.make_async_remote_copy(
                src_ref=red_ref,
                dst_ref=out_ref.at[pl.ds(my_pos * CHUNK, CHUNK), :],
                send_sem=send2.at[d],
                recv_sem=recv2.at[d],
                device_id=(tgt,),
                device_id_type=pl.DeviceIdType.MESH,
            )
            rd.start()
            sends2.append(rd)

        out_ref[pl.ds(my_pos * CHUNK, CHUNK), :] = red_ref[...]
        for d in range(1, N_DEV):
            sends2[d - 1].wait_recv()
        for rd in sends1:
            rd.wait_send()
        for rd in sends2:
            rd.wait_send()

    out = pl.pallas_call(
        body,
        out_shape=jax.ShapeDtypeStruct((ROWS, D), jnp.float32),
        in_specs=[pl.BlockSpec(memory_space=pltpu.VMEM)] * 5,
        out_specs=pl.BlockSpec(memory_space=pltpu.VMEM),
        scratch_shapes=[
            pltpu.VMEM((ROWS, D), jnp.float32),
            pltpu.VMEM((ROWS, D), jnp.float32),
            pltpu.VMEM((CHUNK, D), jnp.float32),
            pltpu.VMEM((N_DEV, CHUNK, D), jnp.float32),
            pltpu.SemaphoreType.DMA((N_DEV,)),
            pltpu.SemaphoreType.DMA((N_DEV,)),
            pltpu.SemaphoreType.DMA((N_DEV,)),
            pltpu.SemaphoreType.DMA((N_DEV,)),
        ],
        compiler_params=pltpu.CompilerParams(collective_id=0),
    )(x2, Wq, Wo, K_loc, V_loc)
    return out.reshape(B, SQ, D)
